# baseline (speedup 1.0000x reference)
import functools

import jax
import jax.numpy as jnp
from jax import lax
from jax.experimental import pallas as pl
from jax.experimental.pallas import tpu as pltpu
from jax.experimental.pallas import tpu_sc as plsc

N = 10000
E = 320000
D = 128
D_OUT = 512

NC = 2
NS = 16
NW = NC * NS
CHUNK = 128
NBUF = 2
NCHUNKH = 40
NCHUNK = 2 * NCHUNKH
EPW = NCHUNK * CHUNK
EP = NW * EPW
DUMMY = N
NPAD = N + 8
RPT = 632
RPT_LAST = N - (NS - 1) * RPT


def _sc_aggregate(x, col_w, row_w):
    mesh = plsc.VectorSubcoreMesh(core_axis_name="c", subcore_axis_name="s")

    @functools.partial(
        pl.kernel,
        out_type=jax.ShapeDtypeStruct((NC, N, D), jnp.float32),
        mesh=mesh,
        scratch_types=[
            pltpu.VMEM((NCHUNK, CHUNK), jnp.int32),
            pltpu.VMEM((NCHUNKH, CHUNK), jnp.int32),
            pltpu.VMEM((NBUF, CHUNK, D), jnp.float32),
            pltpu.VMEM_SHARED((NPAD, D), jnp.float32),
            pltpu.SemaphoreType.DMA((NBUF,)),
        ],
    )
    def sc_kernel(x_hbm, col_hbm, row_hbm, out_hbm, col_v, row_v, rows_v,
                  acc_sh, gsem):
        c = lax.axis_index("c")
        s = lax.axis_index("s")
        wid = c * NS + s

        pltpu.sync_copy(col_hbm.at[wid], col_v)
        pltpu.sync_copy(row_hbm.at[wid, 0], row_v)

        @pl.when(s < NS - 1)
        def _():
            pltpu.sync_copy(x_hbm.at[pl.ds(s * RPT, RPT)],
                            acc_sh.at[pl.ds(s * RPT, RPT)])

        @pl.when(s == NS - 1)
        def _():
            pltpu.sync_copy(x_hbm.at[pl.ds((NS - 1) * RPT, RPT_LAST)],
                            acc_sh.at[pl.ds((NS - 1) * RPT, RPT_LAST)])

        plsc.subcore_barrier()

        def emit_half(h, jbase_of):
            def pair(i, carry):
                j0 = jbase_of(i)
                d0 = pltpu.async_copy(x_hbm.at[col_v.at[j0]], rows_v.at[0],
                                      gsem.at[0])
                d1 = pltpu.async_copy(x_hbm.at[col_v.at[j0 + 1]],
                                      rows_v.at[1], gsem.at[1])
                d0.wait()
                pltpu.sync_copy(rows_v.at[0],
                                acc_sh.at[row_v.at[2 * i]], add=True)
                d1.wait()
                pltpu.sync_copy(rows_v.at[1],
                                acc_sh.at[row_v.at[2 * i + 1]], add=True)
                return carry

            lax.fori_loop(0, NCHUNKH // 2, pair, 0)

        emit_half(0, lambda i: 2 * i)
        pltpu.sync_copy(row_hbm.at[wid, 1], row_v)
        emit_half(1, lambda i: NCHUNKH + 2 * i)
        plsc.subcore_barrier()

        @pl.when(s < NS - 1)
        def _():
            pltpu.sync_copy(acc_sh.at[pl.ds(s * RPT, RPT)],
                            out_hbm.at[c, pl.ds(s * RPT, RPT)])

        @pl.when(s == NS - 1)
        def _():
            pltpu.sync_copy(acc_sh.at[pl.ds((NS - 1) * RPT, RPT_LAST)],
                            out_hbm.at[c, pl.ds((NS - 1) * RPT, RPT_LAST)])

    return sc_kernel(x, col_w, row_w)


def _combine_matmul(x, acc, W, b):
    BLK = 1000
    grid = N // BLK

    def tc_kernel(x_ref, a0_ref, a1_ref, w_ref, b_ref, o_ref):
        sm = a0_ref[...] + a1_ref[...] - x_ref[...]
        o_ref[...] = lax.dot_general(
            sm, w_ref[...], (((1,), (1,)), ((), ())),
            preferred_element_type=jnp.float32) + b_ref[...]

    return pl.pallas_call(
        tc_kernel,
        grid=(grid,),
        in_specs=[
            pl.BlockSpec((BLK, D), lambda i: (i, 0)),
            pl.BlockSpec((BLK, D), lambda i: (i, 0)),
            pl.BlockSpec((BLK, D), lambda i: (i, 0)),
            pl.BlockSpec((D_OUT, D), lambda i: (0, 0)),
            pl.BlockSpec((1, D_OUT), lambda i: (0, 0)),
        ],
        out_specs=pl.BlockSpec((BLK, D_OUT), lambda i: (i, 0)),
        out_shape=jax.ShapeDtypeStruct((N, D_OUT), jnp.float32),
    )(x, acc[0], acc[1], W, b.reshape(1, D_OUT))


def kernel(x, edge_index, W, b):
    ei = edge_index.astype(jnp.int32)
    row = ei[0]
    col = ei[1]
    pad = EP - E
    row_w = jnp.concatenate(
        [row, jnp.full((pad,), DUMMY, jnp.int32)]).reshape(
            NW, 2, NCHUNKH, CHUNK)
    col_w = jnp.concatenate(
        [col, jnp.zeros((pad,), jnp.int32)]).reshape(NW, NCHUNK, CHUNK)
    acc = _sc_aggregate(x, col_w, row_w)
    return _combine_matmul(x, acc, W, b)


# speedup vs baseline: 1.5007x; 1.5007x over previous
import functools

import jax
import jax.numpy as jnp
from jax import lax
from jax.experimental import pallas as pl
from jax.experimental.pallas import tpu as pltpu
from jax.experimental.pallas import tpu_sc as plsc

N = 10000
E = 320000
D = 128
D_OUT = 512

NC = 2
NS = 16
NW = NC * NS
CHUNK = 128
NCHUNK = -(-E // (NW * CHUNK))
EPW = NCHUNK * CHUNK
EP = NW * EPW
DUMMY = N
NPAD = N + 8
RPT = 632
RPT_LAST = N - (NS - 1) * RPT


def _sc_aggregate(x, col_w, row_w):
    mesh = plsc.VectorSubcoreMesh(core_axis_name="c", subcore_axis_name="s")

    @functools.partial(
        pl.kernel,
        out_type=jax.ShapeDtypeStruct((NC, N, D), jnp.float32),
        mesh=mesh,
        scratch_types=[
            pltpu.VMEM((NCHUNK, CHUNK), jnp.int32),
            pltpu.VMEM((NCHUNK, CHUNK), jnp.int32),
            pltpu.VMEM((CHUNK, D), jnp.float32),
            pltpu.VMEM_SHARED((NPAD, D), jnp.float32),
            pltpu.SemaphoreType.DMA,
        ],
    )
    def sc_kernel(x_hbm, col_hbm, row_hbm, out_hbm, col_v, row_v, rows_v,
                  acc_sh, sem):
        c = lax.axis_index("c")
        s = lax.axis_index("s")
        wid = c * NS + s

        pltpu.sync_copy(col_hbm.at[wid], col_v)
        pltpu.sync_copy(row_hbm.at[wid], row_v)

        @pl.when(s < NS - 1)
        def _():
            pltpu.sync_copy(x_hbm.at[pl.ds(s * RPT, RPT)],
                            acc_sh.at[pl.ds(s * RPT, RPT)])

        @pl.when(s == NS - 1)
        def _():
            pltpu.sync_copy(x_hbm.at[pl.ds((NS - 1) * RPT, RPT_LAST)],
                            acc_sh.at[pl.ds((NS - 1) * RPT, RPT_LAST)])

        plsc.subcore_barrier()

        def body(j, carry):
            pltpu.async_copy(x_hbm.at[col_v.at[j]], rows_v, sem).wait()
            pltpu.sync_copy(rows_v, acc_sh.at[row_v.at[j]], add=True)
            return carry

        lax.fori_loop(0, NCHUNK, body, 0)
        plsc.subcore_barrier()

        @pl.when(s < NS - 1)
        def _():
            pltpu.sync_copy(acc_sh.at[pl.ds(s * RPT, RPT)],
                            out_hbm.at[c, pl.ds(s * RPT, RPT)])

        @pl.when(s == NS - 1)
        def _():
            pltpu.sync_copy(acc_sh.at[pl.ds((NS - 1) * RPT, RPT_LAST)],
                            out_hbm.at[c, pl.ds((NS - 1) * RPT, RPT_LAST)])

    return sc_kernel(x, col_w, row_w)


def _combine_matmul(x, acc, W, b):
    BLK = 1000
    grid = N // BLK

    def tc_kernel(x_ref, a0_ref, a1_ref, w_ref, b_ref, o_ref):
        sm = a0_ref[...] + a1_ref[...] - x_ref[...]
        o_ref[...] = lax.dot_general(
            sm, w_ref[...], (((1,), (1,)), ((), ())),
            preferred_element_type=jnp.float32) + b_ref[...]

    return pl.pallas_call(
        tc_kernel,
        grid=(grid,),
        in_specs=[
            pl.BlockSpec((BLK, D), lambda i: (i, 0)),
            pl.BlockSpec((BLK, D), lambda i: (i, 0)),
            pl.BlockSpec((BLK, D), lambda i: (i, 0)),
            pl.BlockSpec((D_OUT, D), lambda i: (0, 0)),
            pl.BlockSpec((1, D_OUT), lambda i: (0, 0)),
        ],
        out_specs=pl.BlockSpec((BLK, D_OUT), lambda i: (i, 0)),
        out_shape=jax.ShapeDtypeStruct((N, D_OUT), jnp.float32),
    )(x, acc[0], acc[1], W, b.reshape(1, D_OUT))


def kernel(x, edge_index, W, b):
    ei = edge_index.astype(jnp.int32)
    row = ei[0]
    col = ei[1]
    pad = EP - E
    row_w = jnp.concatenate(
        [row, jnp.full((pad,), DUMMY, jnp.int32)]).reshape(NW, NCHUNK, CHUNK)
    col_w = jnp.concatenate(
        [col, jnp.zeros((pad,), jnp.int32)]).reshape(NW, NCHUNK, CHUNK)
    acc = _sc_aggregate(x, col_w, row_w)
    return _combine_matmul(x, acc, W, b)
